# baseline (speedup 1.0000x reference)
import functools

import jax
import jax.numpy as jnp
from jax import lax
from jax.experimental import pallas as pl
from jax.experimental.pallas import tpu as pltpu
from jax.experimental.pallas import tpu_sc as plsc

F_TOTAL = 4
N_ROWS = 100000
HALF = N_ROWS // 2
DIM = 64
PAIR = 2 * DIM
BATCH = 16384
NC, NS = 2, 16
NW = NC * NS
BPW = BATCH // NW
R = 64
NCH = BPW // R
LANE = 16
DC = DIM // LANE
BMIX = 2000


def _mix_body(w_ref, f_hbm, out_ref, vbuf, sem):
    i = pl.program_id(0)

    @pl.when(i == 0)
    def _prologue():
        pltpu.make_async_copy(
            f_hbm.at[:, pl.ds(0, BMIX), :], vbuf.at[0], sem.at[0]).start()

    @pl.when(i + 1 < N_ROWS // BMIX)
    def _prefetch():
        pltpu.make_async_copy(
            f_hbm.at[:, pl.ds((i + 1) * BMIX, BMIX), :],
            vbuf.at[(i + 1) % 2], sem.at[(i + 1) % 2]).start()

    pltpu.make_async_copy(
        f_hbm.at[:, pl.ds(i * BMIX, BMIX), :],
        vbuf.at[i % 2], sem.at[i % 2]).wait()
    x = vbuf[i % 2]
    mixed = (w_ref[0] * x[0] + w_ref[1] * x[1]
             + w_ref[2] * x[2] + w_ref[3] * x[3])
    y = mixed.reshape(BMIX // 2, 2, DIM)
    out_ref[...] = jnp.concatenate([y[:, 0, :], y[:, 1, :]], axis=1)


_mix = pl.pallas_call(
    _mix_body,
    grid=(N_ROWS // BMIX,),
    in_specs=[
        pl.BlockSpec(memory_space=pltpu.SMEM),
        pl.BlockSpec(memory_space=pltpu.HBM),
    ],
    out_specs=pl.BlockSpec((BMIX // 2, PAIR), lambda i: (i, 0)),
    out_shape=jax.ShapeDtypeStruct((HALF, PAIR), jnp.float32),
    scratch_shapes=[
        pltpu.VMEM((2, F_TOTAL, BMIX, DIM), jnp.float32),
        pltpu.SemaphoreType.DMA((2,)),
    ],
)


def _score_body(t_hbm, idx_i_hbm, idx_j_hbm, par_hbm, out_hbm,
                par_v, idx_v, off_v, gbuf, sq_v, out_v, sem):
    wid = lax.axis_index("s") * NC + lax.axis_index("c")
    base = wid * BPW

    pltpu.sync_copy(par_hbm, par_v)
    lanes = lax.iota(jnp.int32, LANE)
    b_vec = par_v[pl.ds(0, LANE)]
    k_vec = par_v[pl.ds(LANE, LANE)]

    pltpu.sync_copy(idx_i_hbm.at[pl.ds(base, BPW)], idx_v.at[0])
    pltpu.sync_copy(idx_j_hbm.at[pl.ds(base, BPW)], idx_v.at[1])

    def idx_body(v, carry):
        sl = pl.ds(v * LANE, LANE)
        for side in range(2):
            n = idx_v[side, sl]
            off_v[side, sl] = (n & 1) * DIM
            idx_v[side, sl] = n >> 1
        return carry

    lax.fori_loop(0, BPW // LANE, idx_body, 0)

    def chunk_body(k, carry):
        copies = []
        for side in range(2):
            cp = pltpu.make_async_copy(
                t_hbm.at[idx_v.at[side, pl.ds(k * R, R)]],
                gbuf.at[side], sem)
            cp.start()
            copies.append(cp)
        for cp in copies:
            cp.wait()

        def blk_body(blk, carry2):
            rbase = k * R + blk * LANE
            oiv = off_v[0, pl.ds(rbase, LANE)]
            ojv = off_v[1, pl.ds(rbase, LANE)]
            for rl in range(LANE):
                r = blk * LANE + rl
                oi = oiv[rl]
                oj = ojv[rl]
                sq = None
                for c in range(DC):
                    a = (gbuf[0, r, pl.ds(oi + c * LANE, LANE)]
                         - gbuf[1, r, pl.ds(oj + c * LANE, LANE)])
                    sq = a * a if sq is None else sq + a * a
                sq_v[pl.ds(rl * LANE, LANE)] = sq
            o = plsc.load_gather(sq_v, [lanes * LANE])
            for l in range(1, LANE):
                o += plsc.load_gather(sq_v, [lanes * LANE + l])
            logit = b_vec - k_vec * o
            eneg = jnp.exp(-jnp.abs(logit))
            inv = 1.0 / (1.0 + eneg)
            res = jnp.where(logit >= 0.0, inv, eneg * inv)
            out_v[pl.ds(rbase, LANE)] = res
            return carry2

        lax.fori_loop(0, R // LANE, blk_body, 0)
        return carry

    lax.fori_loop(0, NCH, chunk_body, 0)
    pltpu.sync_copy(out_v, out_hbm.at[pl.ds(base, BPW)])


_score = functools.partial(
    pl.kernel,
    out_type=jax.ShapeDtypeStruct((BATCH,), jnp.float32),
    mesh=plsc.VectorSubcoreMesh(core_axis_name="c", subcore_axis_name="s"),
    compiler_params=pltpu.CompilerParams(needs_layout_passes=False),
    scratch_types=[
        pltpu.VMEM((2 * LANE,), jnp.float32),
        pltpu.VMEM((2, BPW), jnp.int32),
        pltpu.VMEM((2, BPW), jnp.int32),
        pltpu.VMEM((2, R, PAIR), jnp.float32),
        pltpu.VMEM((LANE * LANE,), jnp.float32),
        pltpu.VMEM((BPW,), jnp.float32),
        pltpu.SemaphoreType.DMA,
    ],
)(_score_body)


@jax.jit
def kernel(features, feature_weights, intercept, slope, idx_i, idx_j):
    w4 = jax.nn.softmax(feature_weights.reshape(-1).astype(jnp.float32))
    packed = _mix(w4, features)
    par = jnp.concatenate([
        jnp.full((LANE,), intercept, dtype=jnp.float32),
        jnp.full((LANE,), slope, dtype=jnp.float32),
    ])
    return _score(packed, idx_i, idx_j, par)


# speedup vs baseline: 1.0720x; 1.0720x over previous
import functools

import jax
import jax.numpy as jnp
from jax import lax
from jax.experimental import pallas as pl
from jax.experimental.pallas import tpu as pltpu
from jax.experimental.pallas import tpu_sc as plsc

F_TOTAL = 4
N_ROWS = 100000
HALF = N_ROWS // 2
DIM = 64
PAIR = 2 * DIM
BATCH = 16384
NC, NS = 2, 16
NW = NC * NS
BPW = BATCH // NW
R = 64
NCH = BPW // R
LANE = 16
DC = DIM // LANE
BMIX = 2000


def _mix_body(w_ref, fa_ref, fb_ref, out_ref):
    xa = fa_ref[...]
    xb = fb_ref[...]
    ma = w_ref[0] * xa[0] + w_ref[1] * xa[1] + w_ref[2] * xa[2] \
        + w_ref[3] * xa[3]
    mb = w_ref[0] * xb[0] + w_ref[1] * xb[1] + w_ref[2] * xb[2] \
        + w_ref[3] * xb[3]
    out_ref[...] = jnp.concatenate([ma, mb], axis=1)


_mix = pl.pallas_call(
    _mix_body,
    grid=(HALF // BMIX,),
    in_specs=[
        pl.BlockSpec(memory_space=pltpu.SMEM),
        pl.BlockSpec((F_TOTAL, BMIX, DIM), lambda i: (0, i, 0)),
        pl.BlockSpec((F_TOTAL, BMIX, DIM),
                     lambda i: (0, i + HALF // BMIX, 0)),
    ],
    out_specs=pl.BlockSpec((BMIX, PAIR), lambda i: (i, 0)),
    out_shape=jax.ShapeDtypeStruct((HALF, PAIR), jnp.float32),
)


def _score_body(t_hbm, idx_i_hbm, idx_j_hbm, par_hbm, out_hbm,
                par_v, idx_v, off_v, gbuf, sq_v, out_v, sem):
    wid = lax.axis_index("s") * NC + lax.axis_index("c")
    base = wid * BPW

    pltpu.sync_copy(par_hbm, par_v)
    lanes = lax.iota(jnp.int32, LANE)
    b_vec = par_v[pl.ds(0, LANE)]
    k_vec = par_v[pl.ds(LANE, LANE)]

    pltpu.sync_copy(idx_i_hbm.at[pl.ds(base, BPW)], idx_v.at[0])
    pltpu.sync_copy(idx_j_hbm.at[pl.ds(base, BPW)], idx_v.at[1])

    def idx_body(v, carry):
        sl = pl.ds(v * LANE, LANE)
        for side in range(2):
            n = idx_v[side, sl]
            hi = n >= HALF
            off_v[side, sl] = jnp.where(hi, DIM, 0)
            idx_v[side, sl] = n - jnp.where(hi, HALF, 0)
        return carry

    lax.fori_loop(0, BPW // LANE, idx_body, 0)

    def chunk_body(k, carry):
        copies = []
        for side in range(2):
            cp = pltpu.make_async_copy(
                t_hbm.at[idx_v.at[side, pl.ds(k * R, R)]],
                gbuf.at[side], sem)
            cp.start()
            copies.append(cp)
        for cp in copies:
            cp.wait()

        def blk_body(blk, carry2):
            rbase = k * R + blk * LANE
            oiv = off_v[0, pl.ds(rbase, LANE)]
            ojv = off_v[1, pl.ds(rbase, LANE)]
            for rl in range(LANE):
                r = blk * LANE + rl
                oi = oiv[rl]
                oj = ojv[rl]
                sq = None
                for c in range(DC):
                    a = (gbuf[0, r, pl.ds(oi + c * LANE, LANE)]
                         - gbuf[1, r, pl.ds(oj + c * LANE, LANE)])
                    sq = a * a if sq is None else sq + a * a
                sq_v[pl.ds(rl * LANE, LANE)] = sq
            o = plsc.load_gather(sq_v, [lanes * LANE])
            for l in range(1, LANE):
                o += plsc.load_gather(sq_v, [lanes * LANE + l])
            logit = b_vec - k_vec * o
            eneg = jnp.exp(-jnp.abs(logit))
            inv = 1.0 / (1.0 + eneg)
            res = jnp.where(logit >= 0.0, inv, eneg * inv)
            out_v[pl.ds(rbase, LANE)] = res
            return carry2

        lax.fori_loop(0, R // LANE, blk_body, 0)
        return carry

    lax.fori_loop(0, NCH, chunk_body, 0)
    pltpu.sync_copy(out_v, out_hbm.at[pl.ds(base, BPW)])


_score = functools.partial(
    pl.kernel,
    out_type=jax.ShapeDtypeStruct((BATCH,), jnp.float32),
    mesh=plsc.VectorSubcoreMesh(core_axis_name="c", subcore_axis_name="s"),
    compiler_params=pltpu.CompilerParams(needs_layout_passes=False),
    scratch_types=[
        pltpu.VMEM((2 * LANE,), jnp.float32),
        pltpu.VMEM((2, BPW), jnp.int32),
        pltpu.VMEM((2, BPW), jnp.int32),
        pltpu.VMEM((2, R, PAIR), jnp.float32),
        pltpu.VMEM((LANE * LANE,), jnp.float32),
        pltpu.VMEM((BPW,), jnp.float32),
        pltpu.SemaphoreType.DMA,
    ],
)(_score_body)


@jax.jit
def kernel(features, feature_weights, intercept, slope, idx_i, idx_j):
    w4 = jax.nn.softmax(feature_weights.reshape(-1).astype(jnp.float32))
    packed = _mix(w4, features, features)
    par = jnp.concatenate([
        jnp.full((LANE,), intercept, dtype=jnp.float32),
        jnp.full((LANE,), slope, dtype=jnp.float32),
    ])
    return _score(packed, idx_i, idx_j, par)
